# baseline (speedup 1.0000x reference)
import functools

import jax
import jax.numpy as jnp
from jax import lax
from jax.experimental import pallas as pl
from jax.experimental.pallas import tpu as pltpu
from jax.experimental.pallas import tpu_sc as plsc

NUM_CORES = 2
NUM_SUBCORES = 16
NUM_WORKERS = NUM_CORES * NUM_SUBCORES

ROUTE_TILE = 256
RANK_TILE = 128
FF_BLOCK = 1024
FINAL_TILE = 128


def _routerank_body(n_rt, cap, dump, x_ref, wrt_ref, slot_ref, ge_ref,
                    gcol_s, ecol_s, grow_s, erow_s):
    i = pl.program_id(0)
    t = ROUTE_TILE

    @pl.when(i < n_rt)
    def _():
        l = jnp.dot(x_ref[...], wrt_ref[...],
                    preferred_element_type=jnp.float32)
        m = jnp.max(l, axis=1, keepdims=True)
        gate = 1.0 / jnp.sum(jnp.exp(l - m), axis=1, keepdims=True)
        iot = lax.broadcasted_iota(jnp.int32, l.shape, 1)
        big = jnp.int32(l.shape[1])
        eidx = jnp.min(jnp.where(l == m, iot, big), axis=1, keepdims=True)
        gcol_s[pl.ds(i * t, t), :] = gate
        ecol_s[pl.ds(i * t, t), :] = eidx
        grow_s[:, pl.ds(i * t, t)] = gate.reshape(1, t)
        erow_s[:, pl.ds(i * t, t)] = eidx.reshape(1, t)

    @pl.when(i >= n_rt)
    def _():
        j = i - n_rt
        gi = gcol_s[pl.ds(j * t, t), :]
        ei = ecol_s[pl.ds(j * t, t), :]
        gj = grow_s[...]
        ej = erow_s[...]
        shape = (t, gj.shape[1])
        ii = j * t + lax.broadcasted_iota(jnp.int32, shape, 0)
        jj = lax.broadcasted_iota(jnp.int32, shape, 1)
        ahead = (gj > gi) | ((gj == gi) & (jj < ii))
        cnt = ((ej == ei) & ahead).astype(jnp.int32)
        rank = jnp.sum(cnt, axis=1, keepdims=True)
        kept = rank < cap
        slot_ref[...] = jnp.where(kept, ei * cap + rank, dump)
        ge_ref[...] = jnp.where(kept, gi, 0.0)


def _make_routerank(n_tok, d_model, n_exp, cap, dump, interpret=False):
    n_rt = n_tok // ROUTE_TILE
    return pl.pallas_call(
        functools.partial(_routerank_body, n_rt, cap, dump),
        grid=(2 * n_rt,),
        in_specs=[
            pl.BlockSpec((ROUTE_TILE, d_model),
                         lambda i: (jnp.minimum(i, n_rt - 1), 0)),
            pl.BlockSpec((d_model, n_exp), lambda i: (0, 0)),
        ],
        out_specs=[
            pl.BlockSpec((ROUTE_TILE, 1),
                         lambda i: (jnp.maximum(i - n_rt, 0), 0)),
            pl.BlockSpec((ROUTE_TILE, 1),
                         lambda i: (jnp.maximum(i - n_rt, 0), 0)),
        ],
        out_shape=[
            jax.ShapeDtypeStruct((n_tok, 1), jnp.int32),
            jax.ShapeDtypeStruct((n_tok, 1), jnp.float32),
        ],
        scratch_shapes=[
            pltpu.VMEM((n_tok, 1), jnp.float32),
            pltpu.VMEM((n_tok, 1), jnp.int32),
            pltpu.VMEM((1, n_tok), jnp.float32),
            pltpu.VMEM((1, n_tok), jnp.int32),
        ],
        interpret=interpret,
    )


def _ffn_body(n_fb, x_ref, w1_ref, b1_ref, w2_ref, b2_ref, y_ref):
    f = pl.program_id(1)

    h = jnp.dot(x_ref[...], w1_ref[0], preferred_element_type=jnp.float32)
    h = jnp.maximum(h + b1_ref[0], 0.0)
    p = jnp.dot(h, w2_ref[0], preferred_element_type=jnp.float32)

    @pl.when(f == 0)
    def _():
        y_ref[...] = p

    @pl.when(f > 0)
    def _():
        y_ref[...] += p

    @pl.when(f == n_fb - 1)
    def _():
        y_ref[...] += b2_ref[0]


def _make_ffn(n_exp, cap, d_model, d_ff, n_rows, interpret=False):
    n_fb = d_ff // FF_BLOCK
    return pl.pallas_call(
        functools.partial(_ffn_body, n_fb),
        grid=(n_exp, n_fb),
        in_specs=[
            pl.BlockSpec((cap, d_model), lambda e, f: (e, 0)),
            pl.BlockSpec((1, d_model, FF_BLOCK), lambda e, f: (e, 0, f)),
            pl.BlockSpec((1, 1, FF_BLOCK), lambda e, f: (e, 0, f)),
            pl.BlockSpec((1, FF_BLOCK, d_model), lambda e, f: (e, f, 0)),
            pl.BlockSpec((1, 1, d_model), lambda e, f: (e, 0, 0)),
        ],
        out_specs=pl.BlockSpec((cap, d_model), lambda e, f: (e, 0)),
        out_shape=jax.ShapeDtypeStruct((n_rows, d_model), jnp.float32),
        interpret=interpret,
    )


def _final_body(y_ref, ge_ref, out_ref):
    g = ge_ref[...]
    out_ref[...] = jnp.where(g > 0.0, g * y_ref[...], 0.0)


def _make_final(n_tok, d_model, interpret=False):
    grid = n_tok // FINAL_TILE
    return pl.pallas_call(
        _final_body,
        grid=(grid,),
        in_specs=[
            pl.BlockSpec((FINAL_TILE, d_model), lambda i: (i, 0)),
            pl.BlockSpec((FINAL_TILE, 1), lambda i: (i, 0)),
        ],
        out_specs=pl.BlockSpec((FINAL_TILE, d_model), lambda i: (i, 0)),
        out_shape=jax.ShapeDtypeStruct((n_tok, d_model), jnp.float32),
        interpret=interpret,
    )


def _sc_mesh():
    return plsc.VectorSubcoreMesh(
        core_axis_name="c", subcore_axis_name="s",
        num_cores=NUM_CORES, num_subcores=NUM_SUBCORES)


def _make_dispatch(n_tok, d_model, n_rows):
    per_w = n_tok // NUM_WORKERS
    n_ch = 2
    ch = per_w // n_ch

    @functools.partial(
        pl.kernel,
        out_type=jax.ShapeDtypeStruct((n_rows, d_model), jnp.float32),
        mesh=_sc_mesh(),
        scratch_types=[
            pltpu.VMEM((n_ch, ch), jnp.int32),
            pltpu.VMEM((ch, d_model), jnp.float32),
            pltpu.SemaphoreType.DMA,
        ],
    )
    def dispatch(x_hbm, s_hbm, xd_hbm, idx_v, rows_v, sem):
        wid = lax.axis_index("s") * NUM_CORES + lax.axis_index("c")
        pltpu.sync_copy(s_hbm.at[wid], idx_v)
        for c in range(n_ch):
            base = wid * per_w + c * ch
            pltpu.sync_copy(x_hbm.at[pl.ds(base, ch)], rows_v)
            pltpu.async_copy(rows_v, xd_hbm.at[idx_v.at[c]], sem).wait()

    return dispatch


def _make_combine(n_tok, d_model):
    per_w = n_tok // NUM_WORKERS
    n_ch = 2
    ch = per_w // n_ch

    @functools.partial(
        pl.kernel,
        out_type=jax.ShapeDtypeStruct((n_tok, d_model), jnp.float32),
        mesh=_sc_mesh(),
        scratch_types=[
            pltpu.VMEM((n_ch, ch), jnp.int32),
            pltpu.VMEM((ch, d_model), jnp.float32),
            pltpu.SemaphoreType.DMA,
        ],
    )
    def combine(y_hbm, s_hbm, out_hbm, idx_v, rows_v, sem):
        wid = lax.axis_index("s") * NUM_CORES + lax.axis_index("c")
        pltpu.sync_copy(s_hbm.at[wid], idx_v)
        for c in range(n_ch):
            base = wid * per_w + c * ch
            pltpu.async_copy(y_hbm.at[idx_v.at[c]], rows_v, sem).wait()
            pltpu.sync_copy(rows_v, out_hbm.at[pl.ds(base, ch)])

    return combine


def kernel(x, Wr, W1, b1, W2, b2):
    B, S, D = x.shape
    E = Wr.shape[0]
    Dff = W1.shape[2]
    n_tok = B * S
    cap = int(1.25 * n_tok / E)
    dump = E * cap
    n_rows = E * cap + 8

    x_flat = x.reshape(n_tok, D)
    slot, ge = _make_routerank(n_tok, D, E, cap, dump)(x_flat, Wr.T)

    per_w = n_tok // NUM_WORKERS
    s3 = slot.reshape(NUM_WORKERS, 2, per_w // 2)
    xd = _make_dispatch(n_tok, D, n_rows)(x_flat, s3)
    y = _make_ffn(E, cap, D, Dff, n_rows)(
        xd, W1, b1.reshape(E, 1, Dff), W2, b2.reshape(E, 1, D))
    y_tok = _make_combine(n_tok, D)(y, s3)
    out = _make_final(n_tok, D)(y_tok, ge)
    return out.reshape(B, S, D)


# speedup vs baseline: 1.0428x; 1.0428x over previous
import functools

import jax
import jax.numpy as jnp
from jax import lax
from jax.experimental import pallas as pl
from jax.experimental.pallas import tpu as pltpu
from jax.experimental.pallas import tpu_sc as plsc

NUM_CORES = 2
NUM_SUBCORES = 16
NUM_WORKERS = NUM_CORES * NUM_SUBCORES

ROUTE_TILE = 256
RANK_TILE = 128
FF_BLOCK = 2048
FINAL_TILE = 128


def _routerank_body(n_rt, cap, dump, x_ref, wrt_ref, slot_ref, ge_ref,
                    gcol_s, ecol_s, grow_s, erow_s):
    i = pl.program_id(0)
    t = ROUTE_TILE

    @pl.when(i < n_rt)
    def _():
        l = jnp.dot(x_ref[...], wrt_ref[...],
                    preferred_element_type=jnp.float32)
        m = jnp.max(l, axis=1, keepdims=True)
        gate = 1.0 / jnp.sum(jnp.exp(l - m), axis=1, keepdims=True)
        iot = lax.broadcasted_iota(jnp.int32, l.shape, 1)
        big = jnp.int32(l.shape[1])
        eidx = jnp.min(jnp.where(l == m, iot, big), axis=1, keepdims=True)
        gcol_s[pl.ds(i * t, t), :] = gate
        ecol_s[pl.ds(i * t, t), :] = eidx
        grow_s[:, pl.ds(i * t, t)] = gate.reshape(1, t)
        erow_s[:, pl.ds(i * t, t)] = eidx.reshape(1, t)

    @pl.when(i >= n_rt)
    def _():
        j = i - n_rt
        gi = gcol_s[pl.ds(j * t, t), :]
        ei = ecol_s[pl.ds(j * t, t), :]
        gj = grow_s[...]
        ej = erow_s[...]
        shape = (t, gj.shape[1])
        ii = j * t + lax.broadcasted_iota(jnp.int32, shape, 0)
        jj = lax.broadcasted_iota(jnp.int32, shape, 1)
        ahead = (gj > gi) | ((gj == gi) & (jj < ii))
        cnt = ((ej == ei) & ahead).astype(jnp.int32)
        rank = jnp.sum(cnt, axis=1, keepdims=True)
        kept = rank < cap
        slot_ref[...] = jnp.where(kept, ei * cap + rank, dump)
        ge_ref[...] = jnp.where(kept, gi, 0.0)


def _make_routerank(n_tok, d_model, n_exp, cap, dump, interpret=False):
    n_rt = n_tok // ROUTE_TILE
    return pl.pallas_call(
        functools.partial(_routerank_body, n_rt, cap, dump),
        grid=(2 * n_rt,),
        in_specs=[
            pl.BlockSpec((ROUTE_TILE, d_model),
                         lambda i: (jnp.minimum(i, n_rt - 1), 0)),
            pl.BlockSpec((d_model, n_exp), lambda i: (0, 0)),
        ],
        out_specs=[
            pl.BlockSpec((ROUTE_TILE, 1),
                         lambda i: (jnp.maximum(i - n_rt, 0), 0)),
            pl.BlockSpec((ROUTE_TILE, 1),
                         lambda i: (jnp.maximum(i - n_rt, 0), 0)),
        ],
        out_shape=[
            jax.ShapeDtypeStruct((n_tok, 1), jnp.int32),
            jax.ShapeDtypeStruct((n_tok, 1), jnp.float32),
        ],
        scratch_shapes=[
            pltpu.VMEM((n_tok, 1), jnp.float32),
            pltpu.VMEM((n_tok, 1), jnp.int32),
            pltpu.VMEM((1, n_tok), jnp.float32),
            pltpu.VMEM((1, n_tok), jnp.int32),
        ],
        interpret=interpret,
    )


def _ffn_body(n_fb, x_ref, w1_ref, b1_ref, w2_ref, b2_ref, y_ref):
    f = pl.program_id(1)

    h = jnp.dot(x_ref[...], w1_ref[0], preferred_element_type=jnp.float32)
    h = jnp.maximum(h + b1_ref[0], 0.0)
    p = jnp.dot(h, w2_ref[0], preferred_element_type=jnp.float32)

    @pl.when(f == 0)
    def _():
        y_ref[...] = p

    @pl.when(f > 0)
    def _():
        y_ref[...] += p

    @pl.when(f == n_fb - 1)
    def _():
        y_ref[...] += b2_ref[0]


def _make_ffn(n_exp, cap, d_model, d_ff, n_rows, interpret=False):
    n_fb = d_ff // FF_BLOCK
    return pl.pallas_call(
        functools.partial(_ffn_body, n_fb),
        grid=(n_exp, n_fb),
        in_specs=[
            pl.BlockSpec((cap, d_model), lambda e, f: (e, 0)),
            pl.BlockSpec((1, d_model, FF_BLOCK), lambda e, f: (e, 0, f)),
            pl.BlockSpec((1, 1, FF_BLOCK), lambda e, f: (e, 0, f)),
            pl.BlockSpec((1, FF_BLOCK, d_model), lambda e, f: (e, f, 0)),
            pl.BlockSpec((1, 1, d_model), lambda e, f: (e, 0, 0)),
        ],
        out_specs=pl.BlockSpec((cap, d_model), lambda e, f: (e, 0)),
        out_shape=jax.ShapeDtypeStruct((n_rows, d_model), jnp.float32),
        interpret=interpret,
    )


def _final_body(y_ref, ge_ref, out_ref):
    g = ge_ref[...]
    out_ref[...] = jnp.where(g > 0.0, g * y_ref[...], 0.0)


def _make_final(n_tok, d_model, interpret=False):
    grid = n_tok // FINAL_TILE
    return pl.pallas_call(
        _final_body,
        grid=(grid,),
        in_specs=[
            pl.BlockSpec((FINAL_TILE, d_model), lambda i: (i, 0)),
            pl.BlockSpec((FINAL_TILE, 1), lambda i: (i, 0)),
        ],
        out_specs=pl.BlockSpec((FINAL_TILE, d_model), lambda i: (i, 0)),
        out_shape=jax.ShapeDtypeStruct((n_tok, d_model), jnp.float32),
        interpret=interpret,
    )


def _sc_mesh():
    return plsc.VectorSubcoreMesh(
        core_axis_name="c", subcore_axis_name="s",
        num_cores=NUM_CORES, num_subcores=NUM_SUBCORES)


def _make_dispatch(n_tok, d_model, n_rows):
    per_w = n_tok // NUM_WORKERS
    n_ch = 2
    ch = per_w // n_ch

    @functools.partial(
        pl.kernel,
        out_type=jax.ShapeDtypeStruct((n_rows, d_model), jnp.float32),
        mesh=_sc_mesh(),
        scratch_types=[
            pltpu.VMEM((n_ch, ch), jnp.int32),
            pltpu.VMEM((ch, d_model), jnp.float32),
            pltpu.SemaphoreType.DMA,
        ],
    )
    def dispatch(x_hbm, s_hbm, xd_hbm, idx_v, rows_v, sem):
        wid = lax.axis_index("s") * NUM_CORES + lax.axis_index("c")
        pltpu.sync_copy(s_hbm.at[wid], idx_v)
        for c in range(n_ch):
            base = wid * per_w + c * ch
            pltpu.sync_copy(x_hbm.at[pl.ds(base, ch)], rows_v)
            pltpu.async_copy(rows_v, xd_hbm.at[idx_v.at[c]], sem).wait()

    return dispatch


def _make_combine(n_tok, d_model):
    per_w = n_tok // NUM_WORKERS
    n_ch = 2
    ch = per_w // n_ch

    @functools.partial(
        pl.kernel,
        out_type=jax.ShapeDtypeStruct((n_tok, d_model), jnp.float32),
        mesh=_sc_mesh(),
        scratch_types=[
            pltpu.VMEM((n_ch, ch), jnp.int32),
            pltpu.VMEM((ch, d_model), jnp.float32),
            pltpu.SemaphoreType.DMA,
        ],
    )
    def combine(y_hbm, s_hbm, out_hbm, idx_v, rows_v, sem):
        wid = lax.axis_index("s") * NUM_CORES + lax.axis_index("c")
        pltpu.sync_copy(s_hbm.at[wid], idx_v)
        for c in range(n_ch):
            base = wid * per_w + c * ch
            pltpu.async_copy(y_hbm.at[idx_v.at[c]], rows_v, sem).wait()
            pltpu.sync_copy(rows_v, out_hbm.at[pl.ds(base, ch)])

    return combine


def kernel(x, Wr, W1, b1, W2, b2):
    B, S, D = x.shape
    E = Wr.shape[0]
    Dff = W1.shape[2]
    n_tok = B * S
    cap = int(1.25 * n_tok / E)
    dump = E * cap
    n_rows = E * cap + 8

    x_flat = x.reshape(n_tok, D)
    slot, ge = _make_routerank(n_tok, D, E, cap, dump)(x_flat, Wr.T)

    per_w = n_tok // NUM_WORKERS
    s3 = slot.reshape(NUM_WORKERS, 2, per_w // 2)
    xd = _make_dispatch(n_tok, D, n_rows)(x_flat, s3)
    y = _make_ffn(E, cap, D, Dff, n_rows)(
        xd, W1, b1.reshape(E, 1, Dff), W2, b2.reshape(E, 1, D))
    y_tok = _make_combine(n_tok, D)(y, s3)
    out = _make_final(n_tok, D)(y_tok, ge)
    return out.reshape(B, S, D)


# speedup vs baseline: 1.1094x; 1.0639x over previous
import functools

import jax
import jax.numpy as jnp
from jax import lax
from jax.experimental import pallas as pl
from jax.experimental.pallas import tpu as pltpu
from jax.experimental.pallas import tpu_sc as plsc

NUM_CORES = 2
NUM_SUBCORES = 16
NUM_WORKERS = NUM_CORES * NUM_SUBCORES

ROUTE_TILE = 256
RANK_TILE = 128
FF_BLOCK = 2048
FINAL_TILE = 128


def _routerank_body(n_rt, cap, dump, x_ref, wrt_ref, slot_ref, ge_ref,
                    gcol_s, ecol_s, grow_s, erow_s):
    i = pl.program_id(0)
    t = ROUTE_TILE

    @pl.when(i < n_rt)
    def _():
        l = jnp.dot(x_ref[...], wrt_ref[...],
                    preferred_element_type=jnp.float32)
        m = jnp.max(l, axis=1, keepdims=True)
        gate = 1.0 / jnp.sum(jnp.exp(l - m), axis=1, keepdims=True)
        iot = lax.broadcasted_iota(jnp.int32, l.shape, 1)
        big = jnp.int32(l.shape[1])
        eidx = jnp.min(jnp.where(l == m, iot, big), axis=1, keepdims=True)
        gcol_s[pl.ds(i * t, t), :] = gate
        ecol_s[pl.ds(i * t, t), :] = eidx
        grow_s[:, pl.ds(i * t, t)] = gate.reshape(1, t)
        erow_s[:, pl.ds(i * t, t)] = eidx.reshape(1, t)

    @pl.when(i >= n_rt)
    def _():
        j = i - n_rt
        gi = gcol_s[pl.ds(j * t, t), :]
        ei = ecol_s[pl.ds(j * t, t), :]
        gj = grow_s[...]
        ej = erow_s[...]
        shape = (t, gj.shape[1])
        ii = j * t + lax.broadcasted_iota(jnp.int32, shape, 0)
        jj = lax.broadcasted_iota(jnp.int32, shape, 1)
        ahead = (gj > gi) | ((gj == gi) & (jj < ii))
        cnt = ((ej == ei) & ahead).astype(jnp.int32)
        rank = jnp.sum(cnt, axis=1, keepdims=True)
        kept = rank < cap
        slot_ref[...] = jnp.where(kept, ei * cap + rank, dump)
        ge = jnp.where(kept, gi, 0.0)
        ge_ref[...] = jnp.broadcast_to(ge, (t, 128))


def _make_routerank(n_tok, d_model, n_exp, cap, dump, interpret=False):
    n_rt = n_tok // ROUTE_TILE
    return pl.pallas_call(
        functools.partial(_routerank_body, n_rt, cap, dump),
        grid=(2 * n_rt,),
        in_specs=[
            pl.BlockSpec((ROUTE_TILE, d_model),
                         lambda i: (jnp.minimum(i, n_rt - 1), 0)),
            pl.BlockSpec((d_model, n_exp), lambda i: (0, 0)),
        ],
        out_specs=[
            pl.BlockSpec((ROUTE_TILE, 1),
                         lambda i: (jnp.maximum(i - n_rt, 0), 0)),
            pl.BlockSpec((ROUTE_TILE, 128),
                         lambda i: (jnp.maximum(i - n_rt, 0), 0)),
        ],
        out_shape=[
            jax.ShapeDtypeStruct((n_tok, 1), jnp.int32),
            jax.ShapeDtypeStruct((n_tok, 128), jnp.float32),
        ],
        scratch_shapes=[
            pltpu.VMEM((n_tok, 1), jnp.float32),
            pltpu.VMEM((n_tok, 1), jnp.int32),
            pltpu.VMEM((1, n_tok), jnp.float32),
            pltpu.VMEM((1, n_tok), jnp.int32),
        ],
        interpret=interpret,
    )


def _ffn_body(n_exp, n_fb, x_ref, w1_ref, b1_ref, w2_ref, b2_ref, gd_ref,
              y_ref):
    e = pl.program_id(0)
    f = pl.program_id(1)

    @pl.when(e < n_exp)
    def _():
        h = jnp.dot(x_ref[...], w1_ref[0], preferred_element_type=jnp.float32)
        h = jnp.maximum(h + b1_ref[0], 0.0)
        p = jnp.dot(h, w2_ref[0], preferred_element_type=jnp.float32)

        @pl.when(f == 0)
        def _():
            y_ref[...] = p

        @pl.when(f > 0)
        def _():
            y_ref[...] += p

        @pl.when(f == n_fb - 1)
        def _():
            y_ref[...] = (y_ref[...] + b2_ref[0]) * gd_ref[:, 0:1]

    @pl.when((e == n_exp) & (f == 0))
    def _():
        y_ref[...] = jnp.zeros_like(y_ref)


def _make_ffn(n_exp, cap, d_model, d_ff, n_rows, interpret=False):
    n_fb = d_ff // FF_BLOCK
    clamp = n_exp - 1
    return pl.pallas_call(
        functools.partial(_ffn_body, n_exp, n_fb),
        grid=(n_exp + 1, n_fb),
        in_specs=[
            pl.BlockSpec((cap, d_model),
                         lambda e, f: (jnp.minimum(e, clamp), 0)),
            pl.BlockSpec((1, d_model, FF_BLOCK),
                         lambda e, f: (jnp.minimum(e, clamp), 0, f)),
            pl.BlockSpec((1, 1, FF_BLOCK),
                         lambda e, f: (jnp.minimum(e, clamp), 0, f)),
            pl.BlockSpec((1, FF_BLOCK, d_model),
                         lambda e, f: (jnp.minimum(e, clamp), f, 0)),
            pl.BlockSpec((1, 1, d_model),
                         lambda e, f: (jnp.minimum(e, clamp), 0, 0)),
            pl.BlockSpec((cap, 128),
                         lambda e, f: (jnp.minimum(e, clamp), 0)),
        ],
        out_specs=pl.BlockSpec((cap, d_model), lambda e, f: (e, 0)),
        out_shape=jax.ShapeDtypeStruct((n_rows, d_model), jnp.float32),
        interpret=interpret,
    )


def _sc_mesh():
    return plsc.VectorSubcoreMesh(
        core_axis_name="c", subcore_axis_name="s",
        num_cores=NUM_CORES, num_subcores=NUM_SUBCORES)


N_CH = 4
N_BUF = 2


def _make_dispatch(n_tok, d_model, n_rows):
    per_w = n_tok // NUM_WORKERS
    ch = per_w // N_CH

    @functools.partial(
        pl.kernel,
        out_type=[
            jax.ShapeDtypeStruct((n_rows, d_model), jnp.float32),
            jax.ShapeDtypeStruct((n_rows, 128), jnp.float32),
        ],
        mesh=_sc_mesh(),
        scratch_types=[
            pltpu.VMEM((N_CH, ch), jnp.int32),
            pltpu.VMEM((N_CH, ch, 128), jnp.float32),
            pltpu.VMEM((ch, d_model), jnp.float32),
            pltpu.VMEM((ch, d_model), jnp.float32),
            pltpu.SemaphoreType.DMA,
            pltpu.SemaphoreType.DMA,
            pltpu.SemaphoreType.DMA,
            pltpu.SemaphoreType.DMA,
            pltpu.SemaphoreType.DMA,
        ],
    )
    def dispatch(x_hbm, s_hbm, g_hbm, xd_hbm, gd_hbm, idx_v, gv,
                 buf0, buf1, si0, si1, so0, so1, sg):
        wid = lax.axis_index("s") * NUM_CORES + lax.axis_index("c")
        base = wid * per_w
        bufs, sin, sout = [buf0, buf1], [si0, si1], [so0, so1]
        pltpu.sync_copy(s_hbm.at[wid], idx_v)
        pltpu.sync_copy(g_hbm.at[wid], gv)
        hg = []
        for c in range(N_CH):
            hg.append(pltpu.async_copy(gv.at[c], gd_hbm.at[idx_v.at[c]], sg))
        h_in, h_out = {}, {}
        for c in range(N_BUF):
            h_in[c] = pltpu.async_copy(
                x_hbm.at[pl.ds(base + c * ch, ch)], bufs[c], sin[c])
        for c in range(N_CH):
            b = c % N_BUF
            h_in[c].wait()
            h_out[c] = pltpu.async_copy(bufs[b], xd_hbm.at[idx_v.at[c]],
                                        sout[b])
            if c + N_BUF < N_CH:
                h_out[c].wait()
                h_in[c + N_BUF] = pltpu.async_copy(
                    x_hbm.at[pl.ds(base + (c + N_BUF) * ch, ch)], bufs[b],
                    sin[b])
        for c in range(N_CH - N_BUF, N_CH):
            h_out[c].wait()
        for h in hg:
            h.wait()

    return dispatch


def _make_combine(n_tok, d_model):
    per_w = n_tok // NUM_WORKERS
    ch = per_w // N_CH

    @functools.partial(
        pl.kernel,
        out_type=jax.ShapeDtypeStruct((n_tok, d_model), jnp.float32),
        mesh=_sc_mesh(),
        scratch_types=[
            pltpu.VMEM((N_CH, ch), jnp.int32),
            pltpu.VMEM((ch, d_model), jnp.float32),
            pltpu.VMEM((ch, d_model), jnp.float32),
            pltpu.SemaphoreType.DMA,
            pltpu.SemaphoreType.DMA,
            pltpu.SemaphoreType.DMA,
            pltpu.SemaphoreType.DMA,
        ],
    )
    def combine(y_hbm, s_hbm, out_hbm, idx_v, buf0, buf1,
                si0, si1, so0, so1):
        wid = lax.axis_index("s") * NUM_CORES + lax.axis_index("c")
        base = wid * per_w
        bufs, sin, sout = [buf0, buf1], [si0, si1], [so0, so1]
        pltpu.sync_copy(s_hbm.at[wid], idx_v)

        h_in, h_out = {}, {}
        for c in range(N_BUF):
            h_in[c] = pltpu.async_copy(y_hbm.at[idx_v.at[c]], bufs[c], sin[c])
        for c in range(N_CH):
            b = c % N_BUF
            h_in[c].wait()
            h_out[c] = pltpu.async_copy(
                bufs[b], out_hbm.at[pl.ds(base + c * ch, ch)], sout[b])
            if c + N_BUF < N_CH:
                h_out[c].wait()
                h_in[c + N_BUF] = pltpu.async_copy(
                    y_hbm.at[idx_v.at[c + N_BUF]], bufs[b], sin[b])
        for c in range(N_CH - N_BUF, N_CH):
            h_out[c].wait()

    return combine


def kernel(x, Wr, W1, b1, W2, b2):
    B, S, D = x.shape
    E = Wr.shape[0]
    Dff = W1.shape[2]
    n_tok = B * S
    cap = int(1.25 * n_tok / E)
    dump = E * cap
    n_rows = (E + 1) * cap

    x_flat = x.reshape(n_tok, D)
    slot, ge16 = _make_routerank(n_tok, D, E, cap, dump)(x_flat, Wr.T)

    per_w = n_tok // NUM_WORKERS
    ch = per_w // N_CH
    s3 = slot.reshape(NUM_WORKERS, N_CH, ch)
    g4 = ge16.reshape(NUM_WORKERS, N_CH, ch, 128)
    xd, gd = _make_dispatch(n_tok, D, n_rows)(x_flat, s3, g4)
    y = _make_ffn(E, cap, D, Dff, n_rows)(
        xd, W1, b1.reshape(E, 1, Dff), W2, b2.reshape(E, 1, D), gd)
    out = _make_combine(n_tok, D)(y, s3)
    return out.reshape(B, S, D)


# speedup vs baseline: 1.1104x; 1.0009x over previous
import functools

import jax
import jax.numpy as jnp
from jax import lax
from jax.experimental import pallas as pl
from jax.experimental.pallas import tpu as pltpu
from jax.experimental.pallas import tpu_sc as plsc

NUM_CORES = 2
NUM_SUBCORES = 16
NUM_WORKERS = NUM_CORES * NUM_SUBCORES

ROUTE_TILE = 256
FF_BLOCK = 2048


def _routerank_body(n_rt, cap, dump, x_ref, wrt_ref, slot_ref, ge_ref,
                    gcol_s, ecol_s, grow_s, erow_s):
    i = pl.program_id(0)
    t = ROUTE_TILE

    @pl.when(i < n_rt)
    def _():
        l = jnp.dot(x_ref[...], wrt_ref[...],
                    preferred_element_type=jnp.float32)
        m = jnp.max(l, axis=1, keepdims=True)
        gate = 1.0 / jnp.sum(jnp.exp(l - m), axis=1, keepdims=True)
        iot = lax.broadcasted_iota(jnp.int32, l.shape, 1)
        big = jnp.int32(l.shape[1])
        eidx = jnp.min(jnp.where(l == m, iot, big), axis=1, keepdims=True)
        gcol_s[pl.ds(i * t, t), :] = gate
        ecol_s[pl.ds(i * t, t), :] = eidx
        grow_s[:, pl.ds(i * t, t)] = gate.reshape(1, t)
        erow_s[:, pl.ds(i * t, t)] = eidx.reshape(1, t)

    @pl.when(i >= n_rt)
    def _():
        j = i - n_rt
        gi = gcol_s[pl.ds(j * t, t), :]
        ei = ecol_s[pl.ds(j * t, t), :]
        gj = grow_s[...]
        ej = erow_s[...]
        shape = (t, gj.shape[1])
        ii = j * t + lax.broadcasted_iota(jnp.int32, shape, 0)
        jj = lax.broadcasted_iota(jnp.int32, shape, 1)
        ahead = (gj > gi) | ((gj == gi) & (jj < ii))
        cnt = ((ej == ei) & ahead).astype(jnp.int32)
        rank = jnp.sum(cnt, axis=1, keepdims=True)
        kept = rank < cap
        slot_ref[...] = jnp.where(kept, ei * cap + rank, dump)
        ge = jnp.where(kept, gi, 0.0)
        ge_ref[...] = jnp.broadcast_to(ge, (t, 128))


def _make_routerank(n_tok, d_model, n_exp, cap, dump, interpret=False):
    n_rt = n_tok // ROUTE_TILE
    return pl.pallas_call(
        functools.partial(_routerank_body, n_rt, cap, dump),
        grid=(2 * n_rt,),
        in_specs=[
            pl.BlockSpec((ROUTE_TILE, d_model),
                         lambda i: (jnp.minimum(i, n_rt - 1), 0)),
            pl.BlockSpec((d_model, n_exp), lambda i: (0, 0)),
        ],
        out_specs=[
            pl.BlockSpec((ROUTE_TILE, 1),
                         lambda i: (jnp.maximum(i - n_rt, 0), 0)),
            pl.BlockSpec((ROUTE_TILE, 128),
                         lambda i: (jnp.maximum(i - n_rt, 0), 0)),
        ],
        out_shape=[
            jax.ShapeDtypeStruct((n_tok, 1), jnp.int32),
            jax.ShapeDtypeStruct((n_tok, 128), jnp.float32),
        ],
        scratch_shapes=[
            pltpu.VMEM((n_tok, 1), jnp.float32),
            pltpu.VMEM((n_tok, 1), jnp.int32),
            pltpu.VMEM((1, n_tok), jnp.float32),
            pltpu.VMEM((1, n_tok), jnp.int32),
        ],
        interpret=interpret,
    )


def _ffn_body(n_exp, n_fb, x_ref, w1_ref, b1_ref, w2_ref, b2_ref, gd_ref,
              y_ref):
    e = pl.program_id(0)
    f = pl.program_id(1)

    @pl.when(e < n_exp)
    def _():
        h = jnp.dot(x_ref[...], w1_ref[0], preferred_element_type=jnp.float32)
        h = jnp.maximum(h + b1_ref[0], 0.0)
        p = jnp.dot(h, w2_ref[0], preferred_element_type=jnp.float32)

        @pl.when(f == 0)
        def _():
            y_ref[...] = p

        @pl.when(f > 0)
        def _():
            y_ref[...] += p

        @pl.when(f == n_fb - 1)
        def _():
            y_ref[...] = (y_ref[...] + b2_ref[0]) * gd_ref[:, 0:1]

    @pl.when((e == n_exp) & (f == 0))
    def _():
        y_ref[...] = jnp.zeros_like(y_ref)


def _make_ffn(n_exp, cap, d_model, d_ff, n_rows, interpret=False):
    n_fb = d_ff // FF_BLOCK
    clamp = n_exp - 1
    return pl.pallas_call(
        functools.partial(_ffn_body, n_exp, n_fb),
        grid=(n_exp + 1, n_fb),
        in_specs=[
            pl.BlockSpec((cap, d_model),
                         lambda e, f: (jnp.minimum(e, clamp), 0)),
            pl.BlockSpec((1, d_model, FF_BLOCK),
                         lambda e, f: (jnp.minimum(e, clamp), 0, f)),
            pl.BlockSpec((1, 1, FF_BLOCK),
                         lambda e, f: (jnp.minimum(e, clamp), 0, f)),
            pl.BlockSpec((1, FF_BLOCK, d_model),
                         lambda e, f: (jnp.minimum(e, clamp), f, 0)),
            pl.BlockSpec((1, 1, d_model),
                         lambda e, f: (jnp.minimum(e, clamp), 0, 0)),
            pl.BlockSpec((cap, 128),
                         lambda e, f: (jnp.minimum(e, clamp), 0)),
        ],
        out_specs=pl.BlockSpec((cap, d_model), lambda e, f: (e, 0)),
        out_shape=jax.ShapeDtypeStruct((n_rows, d_model), jnp.float32),
        interpret=interpret,
    )


def _sc_mesh():
    return plsc.VectorSubcoreMesh(
        core_axis_name="c", subcore_axis_name="s",
        num_cores=NUM_CORES, num_subcores=NUM_SUBCORES)


N_CH = 4
N_BUF = 2


def _make_dispatch(n_tok, d_model, n_rows):
    per_w = n_tok // NUM_WORKERS
    ch = per_w // N_CH

    @functools.partial(
        pl.kernel,
        out_type=[
            jax.ShapeDtypeStruct((n_rows, d_model), jnp.float32),
            jax.ShapeDtypeStruct((n_rows, 128), jnp.float32),
        ],
        mesh=_sc_mesh(),
        scratch_types=[
            pltpu.VMEM((N_CH, ch), jnp.int32),
            pltpu.VMEM((N_CH, ch, 128), jnp.float32),
            pltpu.VMEM((ch, d_model), jnp.float32),
            pltpu.VMEM((ch, d_model), jnp.float32),
            pltpu.SemaphoreType.DMA,
            pltpu.SemaphoreType.DMA,
            pltpu.SemaphoreType.DMA,
            pltpu.SemaphoreType.DMA,
            pltpu.SemaphoreType.DMA,
        ],
    )
    def dispatch(x_hbm, s_hbm, g_hbm, xd_hbm, gd_hbm, idx_v, gv,
                 buf0, buf1, si0, si1, so0, so1, sg):
        wid = lax.axis_index("s") * NUM_CORES + lax.axis_index("c")
        base = wid * per_w
        bufs, sin, sout = [buf0, buf1], [si0, si1], [so0, so1]
        pltpu.sync_copy(s_hbm.at[wid], idx_v)
        pltpu.sync_copy(g_hbm.at[wid], gv)
        hg = []
        for c in range(N_CH):
            hg.append(pltpu.async_copy(gv.at[c], gd_hbm.at[idx_v.at[c]], sg))
        h_in, h_out = {}, {}
        for c in range(N_BUF):
            h_in[c] = pltpu.async_copy(
                x_hbm.at[pl.ds(base + c * ch, ch)], bufs[c], sin[c])
        for c in range(N_CH):
            b = c % N_BUF
            h_in[c].wait()
            h_out[c] = pltpu.async_copy(bufs[b], xd_hbm.at[idx_v.at[c]],
                                        sout[b])
            if c + N_BUF < N_CH:
                h_out[c].wait()
                h_in[c + N_BUF] = pltpu.async_copy(
                    x_hbm.at[pl.ds(base + (c + N_BUF) * ch, ch)], bufs[b],
                    sin[b])
        for c in range(N_CH - N_BUF, N_CH):
            h_out[c].wait()
        for h in hg:
            h.wait()

    return dispatch


def _make_combine(n_tok, d_model):
    per_w = n_tok // NUM_WORKERS
    ch = per_w // N_CH

    @functools.partial(
        pl.kernel,
        out_type=jax.ShapeDtypeStruct((n_tok, d_model), jnp.float32),
        mesh=_sc_mesh(),
        scratch_types=[
            pltpu.VMEM((N_CH, ch), jnp.int32),
            pltpu.VMEM((ch, d_model), jnp.float32),
            pltpu.VMEM((ch, d_model), jnp.float32),
            pltpu.SemaphoreType.DMA,
            pltpu.SemaphoreType.DMA,
            pltpu.SemaphoreType.DMA,
            pltpu.SemaphoreType.DMA,
        ],
    )
    def combine(y_hbm, s_hbm, out_hbm, idx_v, buf0, buf1,
                si0, si1, so0, so1):
        wid = lax.axis_index("s") * NUM_CORES + lax.axis_index("c")
        base = wid * per_w
        bufs, sin, sout = [buf0, buf1], [si0, si1], [so0, so1]
        pltpu.sync_copy(s_hbm.at[wid], idx_v)

        h_in, h_out = {}, {}
        for c in range(N_BUF):
            h_in[c] = pltpu.async_copy(y_hbm.at[idx_v.at[c]], bufs[c], sin[c])
        for c in range(N_CH):
            b = c % N_BUF
            h_in[c].wait()
            h_out[c] = pltpu.async_copy(
                bufs[b], out_hbm.at[pl.ds(base + c * ch, ch)], sout[b])
            if c + N_BUF < N_CH:
                h_out[c].wait()
                h_in[c + N_BUF] = pltpu.async_copy(
                    y_hbm.at[idx_v.at[c + N_BUF]], bufs[b], sin[b])
        for c in range(N_CH - N_BUF, N_CH):
            h_out[c].wait()

    return combine


def kernel(x, Wr, W1, b1, W2, b2):
    B, S, D = x.shape
    E = Wr.shape[0]
    Dff = W1.shape[2]
    n_tok = B * S
    cap = int(1.25 * n_tok / E)
    dump = E * cap
    n_rows = (E + 1) * cap

    x_flat = x.reshape(n_tok, D)
    slot, ge16 = _make_routerank(n_tok, D, E, cap, dump)(x_flat, Wr.T)

    per_w = n_tok // NUM_WORKERS
    ch = per_w // N_CH
    s3 = slot.reshape(NUM_WORKERS, N_CH, ch)
    g4 = ge16.reshape(NUM_WORKERS, N_CH, ch, 128)
    xd, gd = _make_dispatch(n_tok, D, n_rows)(x_flat, s3, g4)
    y = _make_ffn(E, cap, D, Dff, n_rows)(
        xd, W1, b1.reshape(E, 1, Dff), W2, b2.reshape(E, 1, D), gd)
    out = _make_combine(n_tok, D)(y, s3)
    return out.reshape(B, S, D)
